# baseline (speedup 1.0000x reference)
import functools

import jax
import jax.numpy as jnp
from jax import lax
from jax.experimental import pallas as pl
from jax.experimental.pallas import tpu as pltpu
from jax.experimental.pallas import tpu_sc as plsc

_NC = 2
_NS = 16
_NW = _NC * _NS
_LANES = 16


def _seg_agg(values, src, dst, ew):
    n, d = values.shape
    e = src.shape[0]
    assert e % _NW == 0
    epw0 = e // _NW
    def _nch(c):
        k = -(-epw0 // c)
        return -(-k // 6) * 6
    def _fits(c):
        per_sub = 3 * c * d + 3 * _nch(c) * c
        return n * d + _NS * per_sub <= 2_090_000
    c_len = next(c for c in range(128, 7, -8) if _fits(c))
    nchunk = _nch(c_len)
    epw = nchunk * c_len
    assert nchunk >= 8
    npt_lo = (n // _NS) // 16 * 16
    tail_rows = n - npt_lo * (_NS - 1)
    assert tail_rows % 16 == 0 and tail_rows > 0

    mesh = plsc.VectorSubcoreMesh(core_axis_name="c", subcore_axis_name="s",
                                  num_cores=_NC, num_subcores=_NS)

    @functools.partial(
        pl.kernel,
        out_type=jax.ShapeDtypeStruct((_NC, n, d), jnp.float32),
        mesh=mesh,
        compiler_params=pltpu.CompilerParams(needs_layout_passes=False,
                                             use_tc_tiling_on_sc=False),
        scratch_types=[
            pltpu.VMEM((nchunk, c_len), jnp.int32),
            pltpu.VMEM((nchunk, c_len), jnp.int32),
            pltpu.VMEM((nchunk, c_len), jnp.float32),
            pltpu.VMEM((3, c_len, d), jnp.float32),
            pltpu.VMEM_SHARED((n, d), jnp.float32),
            pltpu.SemaphoreType.DMA,
            pltpu.SemaphoreType.DMA,
            pltpu.SemaphoreType.DMA,
            pltpu.SemaphoreType.DMA,
            pltpu.SemaphoreType.DMA,
            pltpu.SemaphoreType.DMA,
        ],
    )
    def agg(vals_h, src_h, dst_h, ew_h, zeros_h, out_h, srcv, dstv, eww,
            rin, acc, g0, g1, g2, s0, s1, s2):
        cid = lax.axis_index("c")
        sid = lax.axis_index("s")
        wid = cid * _NS + sid
        stripe_base = sid * npt_lo
        gsem = (g0, g1, g2)
        ssem = (s0, s1, s2)

        def gather_issue(ci, slot):
            pltpu.async_copy(vals_h.at[srcv.at[ci]], rin.at[slot],
                             gsem[slot])

        def gather_wait(slot):
            pltpu.make_async_copy(vals_h.at[srcv.at[0]], rin.at[slot],
                                  gsem[slot]).wait()

        def scat_issue(ci, slot):
            pltpu.async_copy(rin.at[slot], acc.at[dstv.at[ci]], ssem[slot],
                             add=True)

        def scat_wait(slot):
            pltpu.make_async_copy(rin.at[slot], acc.at[dstv.at[0]],
                                  ssem[slot]).wait()

        def scale(ci, slot):
            def one(ei, _):
                bc = plsc.load_gather(
                    eww, [jnp.full((_LANES,), ci, jnp.int32),
                          jnp.full((_LANES,), ei, jnp.int32)])
                for q in range(d // _LANES):
                    sl = pl.ds(q * _LANES, _LANES)
                    rin[slot, ei, sl] = rin[slot, ei, sl] * bc
                return 0
            lax.fori_loop(0, c_len, one, 0)

        pltpu.sync_copy(src_h.at[wid], srcv)
        pltpu.sync_copy(dst_h.at[wid], dstv)
        pltpu.sync_copy(ew_h.at[wid], eww)

        @pl.when(sid < _NS - 1)
        def _():
            pltpu.sync_copy(zeros_h.at[pl.ds(0, npt_lo)],
                            acc.at[pl.ds(stripe_base, npt_lo)])

        @pl.when(sid == _NS - 1)
        def _():
            pltpu.sync_copy(zeros_h, acc.at[pl.ds(stripe_base, tail_rows)])
        plsc.subcore_barrier()

        gather_issue(jnp.int32(0), 0)

        def group(k, _):
            for j in range(6):
                s = j % 3
                i = 6 * k + j
                gather_wait(s)

                @pl.when(i >= 2)
                def _():
                    scat_wait((s + 1) % 3)

                @pl.when(i + 1 < nchunk)
                def _():
                    gather_issue(i + 1, (s + 1) % 3)
                scale(i, s)
                scat_issue(i, s)
            return 0
        lax.fori_loop(0, nchunk // 6, group, 0)
        scat_wait((nchunk - 2) % 3)
        scat_wait((nchunk - 1) % 3)
        plsc.subcore_barrier()

        @pl.when(sid < _NS - 1)
        def _():
            pltpu.sync_copy(acc.at[pl.ds(stripe_base, npt_lo)],
                            out_h.at[cid, pl.ds(stripe_base, npt_lo)])

        @pl.when(sid == _NS - 1)
        def _():
            pltpu.sync_copy(acc.at[pl.ds(stripe_base, tail_rows)],
                            out_h.at[cid, pl.ds(stripe_base, tail_rows)])

    pad = epw - epw0
    def _shape3(a):
        a2 = a.reshape(_NW, epw0)
        if pad:
            a2 = jnp.pad(a2, ((0, 0), (0, pad)))
        return a2.reshape(_NW, nchunk, c_len)
    zeros = jnp.zeros((tail_rows, d), jnp.float32)
    return agg(values, _shape3(src), _shape3(dst), _shape3(ew), zeros)


_BM = 1000


def _row_spec(d):
    return pl.BlockSpec((_BM, d), lambda i: (i, 0))


def _full_spec(shape):
    nd = len(shape)
    return pl.BlockSpec(shape, lambda i, _n=nd: (0,) * _n)


def _stage_a(x, wrelT, wrootT, brel):
    n, d = x.shape

    def body(x_ref, wr, wt, b, y_ref, r_ref):
        xv = x_ref[...]
        x0 = jnp.where(xv == -1.0, 0.0, xv)
        y_ref[...] = jnp.dot(x0, wr[...], preferred_element_type=jnp.float32)
        r_ref[...] = jnp.dot(x0, wt[...], preferred_element_type=jnp.float32) + b[...]

    return pl.pallas_call(
        body,
        grid=(n // _BM,),
        in_specs=[_row_spec(d), _full_spec(wrelT.shape), _full_spec(wrootT.shape),
                  _full_spec(brel.shape)],
        out_specs=[_row_spec(d), _row_spec(d)],
        out_shape=[jax.ShapeDtypeStruct((n, d), jnp.float32)] * 2,
    )(x, wrelT, wrootT, brel)


def _stage_b(sp, r0, wrelT, wrootT, brel):
    n, d = r0.shape

    def body(sa, sb, r0_ref, wr, wt, b, y_ref, r_ref):
        f1 = jax.nn.relu(sa[...] + sb[...] + r0_ref[...])
        y_ref[...] = jnp.dot(f1, wr[...], preferred_element_type=jnp.float32)
        r_ref[...] = jnp.dot(f1, wt[...], preferred_element_type=jnp.float32) + b[...]

    return pl.pallas_call(
        body,
        grid=(n // _BM,),
        in_specs=[_row_spec(d), _row_spec(d), _row_spec(d),
                  _full_spec(wrelT.shape), _full_spec(wrootT.shape), _full_spec(brel.shape)],
        out_specs=[_row_spec(d), _row_spec(d)],
        out_shape=[jax.ShapeDtypeStruct((n, d), jnp.float32)] * 2,
    )(sp[0], sp[1], r0, wrelT, wrootT, brel)


def _stage_c(s1p, r1, sdp, dur_x, gdWrelT, gdWrootT, gdb, gcWrelT, wcombT, bcomb):
    n, d = r1.shape
    dd = dur_x.shape[1]

    def body(sa, sb, r1_ref, da, db, dx, gwr, gwt, gb, cwr, cwt, cb, y_ref, r_ref):
        f2 = jax.nn.relu(sa[...] + sb[...] + r1_ref[...])
        dagg = da[...] + db[...]
        dvec = jax.nn.relu(
            jnp.dot(dagg, gwr[...], preferred_element_type=jnp.float32)
            + jnp.dot(dx[...], gwt[...], preferred_element_type=jnp.float32)
            + gb[...])
        h = jnp.concatenate([f2, dvec], axis=1)
        y_ref[...] = jnp.dot(h, cwr[...], preferred_element_type=jnp.float32)
        r_ref[...] = jnp.dot(h, cwt[...], preferred_element_type=jnp.float32) + cb[...]

    return pl.pallas_call(
        body,
        grid=(n // _BM,),
        in_specs=[_row_spec(d), _row_spec(d), _row_spec(d),
                  _row_spec(dd), _row_spec(dd), _row_spec(dd),
                  _full_spec(gdWrelT.shape), _full_spec(gdWrootT.shape), _full_spec(gdb.shape),
                  _full_spec(gcWrelT.shape), _full_spec(wcombT.shape), _full_spec(bcomb.shape)],
        out_specs=[_row_spec(d), _row_spec(d)],
        out_shape=[jax.ShapeDtypeStruct((n, d), jnp.float32)] * 2,
    )(s1p[0], s1p[1], r1, sdp[0], sdp[1], dur_x,
      gdWrelT, gdWrootT, gdb, gcWrelT, wcombT, bcomb)


def _stage_d(s2p, r2):
    n, d = r2.shape

    def body(sa, sb, r2_ref, ps_ref):
        g = jax.nn.relu(sa[...] + sb[...] + r2_ref[...])
        blk = jnp.sum(g, axis=0, keepdims=True)

        @pl.when(pl.program_id(0) == 0)
        def _():
            ps_ref[...] = blk

        @pl.when(pl.program_id(0) != 0)
        def _():
            ps_ref[...] = ps_ref[...] + blk

    return pl.pallas_call(
        body,
        grid=(n // _BM,),
        in_specs=[_row_spec(d), _row_spec(d), _row_spec(d)],
        out_specs=pl.BlockSpec((1, d), lambda i: (0, 0)),
        out_shape=jax.ShapeDtypeStruct((1, d), jnp.float32),
    )(s2p[0], s2p[1], r2)


def _tail(psum, n_nodes, seqf, fc0WT, fc0b, fccWT, fccb, clsWT, clsb):
    def body(ps, sf, fw, fb, cw, cb, kw, kb, out_ref):
        pooled = ps[...] * (1.0 / n_nodes)
        s = jax.nn.relu(jnp.dot(sf[...], fw[...], preferred_element_type=jnp.float32) + fb[...])
        c = jnp.concatenate([pooled, s], axis=1)
        c = jax.nn.relu(jnp.dot(c, cw[...], preferred_element_type=jnp.float32) + cb[...])
        out_ref[...] = jnp.dot(c, kw[...], preferred_element_type=jnp.float32) + kb[...]

    args = (psum, seqf, fc0WT, fc0b, fccWT, fccb, clsWT, clsb)
    return pl.pallas_call(
        body,
        in_specs=[pl.BlockSpec(a.shape, lambda _nd=a.ndim: (0,) * _nd) for a in args],
        out_specs=pl.BlockSpec((1, clsWT.shape[1]), lambda: (0, 0)),
        out_shape=jax.ShapeDtypeStruct((1, clsWT.shape[1]), jnp.float32),
    )(*args)


def kernel(x, edge_index, edge_attr, dur_x, dur_edge_index, dur_edge_attr,
           sequence_features, g0_Wrel, g0_brel, g0_Wroot, g1_Wrel, g1_brel,
           g1_Wroot, gd0_Wrel, gd0_brel, gd0_Wroot, gc0_Wrel, gc0_brel,
           gc0_Wroot, skip0_W, skip0_b, fc0_W, fc0_b, fcc0_W, fcc0_b,
           cls_W, cls_b):
    n = x.shape[0]
    src = edge_index[0].astype(jnp.int32)
    dst = edge_index[1].astype(jnp.int32)
    dsrc = dur_edge_index[0].astype(jnp.int32)
    ddst = dur_edge_index[1].astype(jnp.int32)
    ea = edge_attr.astype(jnp.float32)
    dea = dur_edge_attr.astype(jnp.float32)

    wcomb = gc0_Wroot + skip0_W
    bcomb = (gc0_brel + skip0_b).reshape(1, -1)

    sdp = _seg_agg(dur_x, dsrc, ddst, dea)

    y0, r0 = _stage_a(x, g0_Wrel.T, g0_Wroot.T, g0_brel.reshape(1, -1))
    s0p = _seg_agg(y0, src, dst, ea)
    y1, r1 = _stage_b(s0p, r0, g1_Wrel.T, g1_Wroot.T, g1_brel.reshape(1, -1))
    s1p = _seg_agg(y1, src, dst, ea)
    y2, r2 = _stage_c(s1p, r1, sdp, dur_x, gd0_Wrel.T, gd0_Wroot.T,
                      gd0_brel.reshape(1, -1), gc0_Wrel.T, wcomb.T, bcomb)
    s2p = _seg_agg(y2, src, dst, ea)
    psum = _stage_d(s2p, r2)
    return _tail(psum, n, sequence_features, fc0_W.T, fc0_b.reshape(1, -1),
                 fcc0_W.T, fcc0_b.reshape(1, -1), cls_W.T, cls_b.reshape(1, -1))


# speedup vs baseline: 1.2674x; 1.2674x over previous
import functools

import jax
import jax.numpy as jnp
from jax import lax
from jax.experimental import pallas as pl
from jax.experimental.pallas import tpu as pltpu
from jax.experimental.pallas import tpu_sc as plsc

_NC = 2
_NS = 16
_NW = _NC * _NS
_LANES = 16


def _seg_agg(values, src, dst, ew):
    n, d = values.shape
    e = src.shape[0]
    assert e % _NW == 0
    epw0 = e // _NW
    def _nch(c):
        k = -(-epw0 // c)
        return k + (k % 2)
    def _fits(c):
        per_sub = 2 * c * d + 3 * _nch(c) * c
        return n * d + _NS * per_sub <= 2_090_000
    c_len = next(c for c in range(128, 7, -8) if _fits(c))
    nchunk = _nch(c_len)
    epw = nchunk * c_len
    assert nchunk >= 8
    npt_lo = (n // _NS) // 16 * 16
    tail_rows = n - npt_lo * (_NS - 1)
    assert tail_rows % 16 == 0 and tail_rows > 0

    mesh = plsc.VectorSubcoreMesh(core_axis_name="c", subcore_axis_name="s",
                                  num_cores=_NC, num_subcores=_NS)

    @functools.partial(
        pl.kernel,
        out_type=jax.ShapeDtypeStruct((_NC, n, d), jnp.float32),
        mesh=mesh,
        compiler_params=pltpu.CompilerParams(needs_layout_passes=False,
                                             use_tc_tiling_on_sc=False),
        scratch_types=[
            pltpu.VMEM((nchunk, c_len), jnp.int32),
            pltpu.VMEM((nchunk, c_len), jnp.int32),
            pltpu.VMEM((nchunk, c_len), jnp.float32),
            pltpu.VMEM((2, c_len, d), jnp.float32),
            pltpu.VMEM_SHARED((n, d), jnp.float32),
            pltpu.SemaphoreType.DMA,
            pltpu.SemaphoreType.DMA,
        ],
    )
    def agg(vals_h, src_h, dst_h, ew_h, zeros_h, out_h, srcv, dstv, eww,
            rin, acc, g0, g1):
        cid = lax.axis_index("c")
        sid = lax.axis_index("s")
        wid = cid * _NS + sid
        stripe_base = sid * npt_lo
        gsem = (g0, g1)

        def gather_issue(ci, slot):
            pltpu.async_copy(vals_h.at[srcv.at[ci]], rin.at[slot],
                             gsem[slot])

        def gather_wait(slot):
            pltpu.make_async_copy(vals_h.at[srcv.at[0]], rin.at[slot],
                                  gsem[slot]).wait()

        def scale(ci, slot):
            def two(p, _):
                ei = 2 * p
                bc0 = plsc.load_gather(
                    eww, [jnp.full((_LANES,), ci, jnp.int32),
                          jnp.full((_LANES,), ei, jnp.int32)])
                bc1 = plsc.load_gather(
                    eww, [jnp.full((_LANES,), ci, jnp.int32),
                          jnp.full((_LANES,), ei + 1, jnp.int32)])
                for q in range(d // _LANES):
                    sl = pl.ds(q * _LANES, _LANES)
                    rin[slot, ei, sl] = rin[slot, ei, sl] * bc0
                    rin[slot, ei + 1, sl] = rin[slot, ei + 1, sl] * bc1
                return 0
            lax.fori_loop(0, c_len // 2, two, 0)

        pltpu.sync_copy(src_h.at[wid], srcv)
        pltpu.sync_copy(dst_h.at[wid], dstv)
        pltpu.sync_copy(ew_h.at[wid], eww)

        @pl.when(sid < _NS - 1)
        def _():
            pltpu.sync_copy(zeros_h.at[pl.ds(0, npt_lo)],
                            acc.at[pl.ds(stripe_base, npt_lo)])

        @pl.when(sid == _NS - 1)
        def _():
            pltpu.sync_copy(zeros_h, acc.at[pl.ds(stripe_base, tail_rows)])
        plsc.subcore_barrier()

        gather_issue(jnp.int32(0), 0)

        def pair(k, _):
            for b in range(2):
                i = 2 * k + b
                gather_wait(b)

                @pl.when(i + 1 < nchunk)
                def _():
                    gather_issue(i + 1, 1 - b)
                scale(i, b)
                pltpu.sync_copy(rin.at[b], acc.at[dstv.at[i]], add=True)
            return 0
        lax.fori_loop(0, nchunk // 2, pair, 0)
        plsc.subcore_barrier()

        @pl.when(sid < _NS - 1)
        def _():
            pltpu.sync_copy(acc.at[pl.ds(stripe_base, npt_lo)],
                            out_h.at[cid, pl.ds(stripe_base, npt_lo)])

        @pl.when(sid == _NS - 1)
        def _():
            pltpu.sync_copy(acc.at[pl.ds(stripe_base, tail_rows)],
                            out_h.at[cid, pl.ds(stripe_base, tail_rows)])

    pad = epw - epw0
    def _shape3(a):
        a2 = a.reshape(_NW, epw0)
        if pad:
            a2 = jnp.pad(a2, ((0, 0), (0, pad)))
        return a2.reshape(_NW, nchunk, c_len)
    zeros = jnp.zeros((tail_rows, d), jnp.float32)
    return agg(values, _shape3(src), _shape3(dst), _shape3(ew), zeros)


_BM = 1000


def _row_spec(d):
    return pl.BlockSpec((_BM, d), lambda i: (i, 0))


def _full_spec(shape):
    nd = len(shape)
    return pl.BlockSpec(shape, lambda i, _n=nd: (0,) * _n)


def _stage_a(x, wrelT, wrootT, brel):
    n, d = x.shape

    def body(x_ref, wr, wt, b, y_ref, r_ref):
        xv = x_ref[...]
        x0 = jnp.where(xv == -1.0, 0.0, xv)
        y_ref[...] = jnp.dot(x0, wr[...], preferred_element_type=jnp.float32)
        r_ref[...] = jnp.dot(x0, wt[...], preferred_element_type=jnp.float32) + b[...]

    return pl.pallas_call(
        body,
        grid=(n // _BM,),
        in_specs=[_row_spec(d), _full_spec(wrelT.shape), _full_spec(wrootT.shape),
                  _full_spec(brel.shape)],
        out_specs=[_row_spec(d), _row_spec(d)],
        out_shape=[jax.ShapeDtypeStruct((n, d), jnp.float32)] * 2,
    )(x, wrelT, wrootT, brel)


def _stage_b(sp, r0, wrelT, wrootT, brel):
    n, d = r0.shape

    def body(sa, sb, r0_ref, wr, wt, b, y_ref, r_ref):
        f1 = jax.nn.relu(sa[...] + sb[...] + r0_ref[...])
        y_ref[...] = jnp.dot(f1, wr[...], preferred_element_type=jnp.float32)
        r_ref[...] = jnp.dot(f1, wt[...], preferred_element_type=jnp.float32) + b[...]

    return pl.pallas_call(
        body,
        grid=(n // _BM,),
        in_specs=[_row_spec(d), _row_spec(d), _row_spec(d),
                  _full_spec(wrelT.shape), _full_spec(wrootT.shape), _full_spec(brel.shape)],
        out_specs=[_row_spec(d), _row_spec(d)],
        out_shape=[jax.ShapeDtypeStruct((n, d), jnp.float32)] * 2,
    )(sp[0], sp[1], r0, wrelT, wrootT, brel)


def _stage_c(s1p, r1, sdp, dur_x, gdWrelT, gdWrootT, gdb, gcWrelT, wcombT, bcomb):
    n, d = r1.shape
    dd = dur_x.shape[1]

    def body(sa, sb, r1_ref, da, db, dx, gwr, gwt, gb, cwr, cwt, cb, y_ref, r_ref):
        f2 = jax.nn.relu(sa[...] + sb[...] + r1_ref[...])
        dagg = da[...] + db[...]
        dvec = jax.nn.relu(
            jnp.dot(dagg, gwr[...], preferred_element_type=jnp.float32)
            + jnp.dot(dx[...], gwt[...], preferred_element_type=jnp.float32)
            + gb[...])
        h = jnp.concatenate([f2, dvec], axis=1)
        y_ref[...] = jnp.dot(h, cwr[...], preferred_element_type=jnp.float32)
        r_ref[...] = jnp.dot(h, cwt[...], preferred_element_type=jnp.float32) + cb[...]

    return pl.pallas_call(
        body,
        grid=(n // _BM,),
        in_specs=[_row_spec(d), _row_spec(d), _row_spec(d),
                  _row_spec(dd), _row_spec(dd), _row_spec(dd),
                  _full_spec(gdWrelT.shape), _full_spec(gdWrootT.shape), _full_spec(gdb.shape),
                  _full_spec(gcWrelT.shape), _full_spec(wcombT.shape), _full_spec(bcomb.shape)],
        out_specs=[_row_spec(d), _row_spec(d)],
        out_shape=[jax.ShapeDtypeStruct((n, d), jnp.float32)] * 2,
    )(s1p[0], s1p[1], r1, sdp[0], sdp[1], dur_x,
      gdWrelT, gdWrootT, gdb, gcWrelT, wcombT, bcomb)


def _stage_d(s2p, r2):
    n, d = r2.shape

    def body(sa, sb, r2_ref, ps_ref):
        g = jax.nn.relu(sa[...] + sb[...] + r2_ref[...])
        blk = jnp.sum(g, axis=0, keepdims=True)

        @pl.when(pl.program_id(0) == 0)
        def _():
            ps_ref[...] = blk

        @pl.when(pl.program_id(0) != 0)
        def _():
            ps_ref[...] = ps_ref[...] + blk

    return pl.pallas_call(
        body,
        grid=(n // _BM,),
        in_specs=[_row_spec(d), _row_spec(d), _row_spec(d)],
        out_specs=pl.BlockSpec((1, d), lambda i: (0, 0)),
        out_shape=jax.ShapeDtypeStruct((1, d), jnp.float32),
    )(s2p[0], s2p[1], r2)


def _tail(psum, n_nodes, seqf, fc0WT, fc0b, fccWT, fccb, clsWT, clsb):
    def body(ps, sf, fw, fb, cw, cb, kw, kb, out_ref):
        pooled = ps[...] * (1.0 / n_nodes)
        s = jax.nn.relu(jnp.dot(sf[...], fw[...], preferred_element_type=jnp.float32) + fb[...])
        c = jnp.concatenate([pooled, s], axis=1)
        c = jax.nn.relu(jnp.dot(c, cw[...], preferred_element_type=jnp.float32) + cb[...])
        out_ref[...] = jnp.dot(c, kw[...], preferred_element_type=jnp.float32) + kb[...]

    args = (psum, seqf, fc0WT, fc0b, fccWT, fccb, clsWT, clsb)
    return pl.pallas_call(
        body,
        in_specs=[pl.BlockSpec(a.shape, lambda _nd=a.ndim: (0,) * _nd) for a in args],
        out_specs=pl.BlockSpec((1, clsWT.shape[1]), lambda: (0, 0)),
        out_shape=jax.ShapeDtypeStruct((1, clsWT.shape[1]), jnp.float32),
    )(*args)


def kernel(x, edge_index, edge_attr, dur_x, dur_edge_index, dur_edge_attr,
           sequence_features, g0_Wrel, g0_brel, g0_Wroot, g1_Wrel, g1_brel,
           g1_Wroot, gd0_Wrel, gd0_brel, gd0_Wroot, gc0_Wrel, gc0_brel,
           gc0_Wroot, skip0_W, skip0_b, fc0_W, fc0_b, fcc0_W, fcc0_b,
           cls_W, cls_b):
    n = x.shape[0]
    src = edge_index[0].astype(jnp.int32)
    dst = edge_index[1].astype(jnp.int32)
    dsrc = dur_edge_index[0].astype(jnp.int32)
    ddst = dur_edge_index[1].astype(jnp.int32)
    ea = edge_attr.astype(jnp.float32)
    dea = dur_edge_attr.astype(jnp.float32)

    wcomb = gc0_Wroot + skip0_W
    bcomb = (gc0_brel + skip0_b).reshape(1, -1)

    sdp = _seg_agg(dur_x, dsrc, ddst, dea)

    y0, r0 = _stage_a(x, g0_Wrel.T, g0_Wroot.T, g0_brel.reshape(1, -1))
    s0p = _seg_agg(y0, src, dst, ea)
    y1, r1 = _stage_b(s0p, r0, g1_Wrel.T, g1_Wroot.T, g1_brel.reshape(1, -1))
    s1p = _seg_agg(y1, src, dst, ea)
    y2, r2 = _stage_c(s1p, r1, sdp, dur_x, gd0_Wrel.T, gd0_Wroot.T,
                      gd0_brel.reshape(1, -1), gc0_Wrel.T, wcomb.T, bcomb)
    s2p = _seg_agg(y2, src, dst, ea)
    psum = _stage_d(s2p, r2)
    return _tail(psum, n, sequence_features, fc0_W.T, fc0_b.reshape(1, -1),
                 fcc0_W.T, fcc0_b.reshape(1, -1), cls_W.T, cls_b.reshape(1, -1))


# speedup vs baseline: 1.2920x; 1.0194x over previous
import functools

import jax
import jax.numpy as jnp
from jax import lax
from jax.experimental import pallas as pl
from jax.experimental.pallas import tpu as pltpu
from jax.experimental.pallas import tpu_sc as plsc

_NC = 2
_NS = 16
_NW = _NC * _NS
_LANES = 16


def _seg_agg(values, src, dst, ew):
    n, d = values.shape
    e = src.shape[0]
    assert e % _NW == 0
    epw0 = e // _NW
    def _nch(c):
        k = -(-epw0 // c)
        return k + (k % 2)
    def _fits(c):
        per_sub = 2 * c * d + 3 * _nch(c) * c
        return n * d + _NS * per_sub <= 2_096_000
    c_len = next(c for c in range(128, 7, -8) if _fits(c))
    nchunk = _nch(c_len)
    epw = nchunk * c_len
    assert nchunk >= 8
    npt_lo = (n // _NS) // 16 * 16
    tail_rows = n - npt_lo * (_NS - 1)
    assert tail_rows % 16 == 0 and tail_rows > 0

    mesh = plsc.VectorSubcoreMesh(core_axis_name="c", subcore_axis_name="s",
                                  num_cores=_NC, num_subcores=_NS)

    @functools.partial(
        pl.kernel,
        out_type=jax.ShapeDtypeStruct((_NC, n, d), jnp.float32),
        mesh=mesh,
        compiler_params=pltpu.CompilerParams(needs_layout_passes=False,
                                             use_tc_tiling_on_sc=False),
        scratch_types=[
            pltpu.VMEM((nchunk, c_len), jnp.int32),
            pltpu.VMEM((nchunk, c_len), jnp.int32),
            pltpu.VMEM((nchunk, c_len), jnp.float32),
            pltpu.VMEM((2, c_len, d), jnp.float32),
            pltpu.VMEM_SHARED((n, d), jnp.float32),
            pltpu.SemaphoreType.DMA,
            pltpu.SemaphoreType.DMA,
        ],
    )
    def agg(vals_h, src_h, dst_h, ew_h, zeros_h, out_h, srcv, dstv, eww,
            rin, acc, g0, g1):
        cid = lax.axis_index("c")
        sid = lax.axis_index("s")
        wid = cid * _NS + sid
        stripe_base = sid * npt_lo
        gsem = (g0, g1)

        def gather_issue(ci, slot):
            pltpu.async_copy(vals_h.at[srcv.at[ci]], rin.at[slot],
                             gsem[slot])

        def gather_wait(slot):
            pltpu.make_async_copy(vals_h.at[srcv.at[0]], rin.at[slot],
                                  gsem[slot]).wait()

        def scale(ci, slot):
            def two(p, _):
                ei = 2 * p
                bc0 = plsc.load_gather(
                    eww, [jnp.full((_LANES,), ci, jnp.int32),
                          jnp.full((_LANES,), ei, jnp.int32)])
                bc1 = plsc.load_gather(
                    eww, [jnp.full((_LANES,), ci, jnp.int32),
                          jnp.full((_LANES,), ei + 1, jnp.int32)])
                for q in range(d // _LANES):
                    sl = pl.ds(q * _LANES, _LANES)
                    rin[slot, ei, sl] = rin[slot, ei, sl] * bc0
                    rin[slot, ei + 1, sl] = rin[slot, ei + 1, sl] * bc1
                return 0
            lax.fori_loop(0, c_len // 2, two, 0)

        pltpu.sync_copy(src_h.at[wid], srcv)
        pltpu.sync_copy(dst_h.at[wid], dstv)
        pltpu.sync_copy(ew_h.at[wid], eww)

        @pl.when(sid < _NS - 1)
        def _():
            pltpu.sync_copy(zeros_h.at[pl.ds(0, npt_lo)],
                            acc.at[pl.ds(stripe_base, npt_lo)])

        @pl.when(sid == _NS - 1)
        def _():
            pltpu.sync_copy(zeros_h, acc.at[pl.ds(stripe_base, tail_rows)])
        plsc.subcore_barrier()

        gather_issue(jnp.int32(0), 0)

        def pair(k, _):
            for b in range(2):
                i = 2 * k + b
                gather_wait(b)

                @pl.when(i + 1 < nchunk)
                def _():
                    gather_issue(i + 1, 1 - b)
                scale(i, b)
                pltpu.sync_copy(rin.at[b], acc.at[dstv.at[i]], add=True)
            return 0
        lax.fori_loop(0, nchunk // 2, pair, 0)
        plsc.subcore_barrier()

        @pl.when(sid < _NS - 1)
        def _():
            pltpu.sync_copy(acc.at[pl.ds(stripe_base, npt_lo)],
                            out_h.at[cid, pl.ds(stripe_base, npt_lo)])

        @pl.when(sid == _NS - 1)
        def _():
            pltpu.sync_copy(acc.at[pl.ds(stripe_base, tail_rows)],
                            out_h.at[cid, pl.ds(stripe_base, tail_rows)])

    pad = epw - epw0
    def _shape3(a):
        a2 = a.reshape(_NW, epw0)
        if pad:
            a2 = jnp.pad(a2, ((0, 0), (0, pad)))
        return a2.reshape(_NW, nchunk, c_len)
    zeros = jnp.zeros((tail_rows, d), jnp.float32)
    return agg(values, _shape3(src), _shape3(dst), _shape3(ew), zeros)


_BM = 1000


def _row_spec(d):
    return pl.BlockSpec((_BM, d), lambda i: (i, 0))


def _full_spec(shape):
    nd = len(shape)
    return pl.BlockSpec(shape, lambda i, _n=nd: (0,) * _n)


def _stage_a(x, wrelT, wrootT, brel):
    n, d = x.shape

    def body(x_ref, wr, wt, b, y_ref, r_ref):
        xv = x_ref[...]
        x0 = jnp.where(xv == -1.0, 0.0, xv)
        y_ref[...] = jnp.dot(x0, wr[...], preferred_element_type=jnp.float32)
        r_ref[...] = jnp.dot(x0, wt[...], preferred_element_type=jnp.float32) + b[...]

    return pl.pallas_call(
        body,
        grid=(n // _BM,),
        in_specs=[_row_spec(d), _full_spec(wrelT.shape), _full_spec(wrootT.shape),
                  _full_spec(brel.shape)],
        out_specs=[_row_spec(d), _row_spec(d)],
        out_shape=[jax.ShapeDtypeStruct((n, d), jnp.float32)] * 2,
    )(x, wrelT, wrootT, brel)


def _stage_b(sp, r0, wrelT, wrootT, brel):
    n, d = r0.shape

    def body(sa, sb, r0_ref, wr, wt, b, y_ref, r_ref):
        f1 = jax.nn.relu(sa[...] + sb[...] + r0_ref[...])
        y_ref[...] = jnp.dot(f1, wr[...], preferred_element_type=jnp.float32)
        r_ref[...] = jnp.dot(f1, wt[...], preferred_element_type=jnp.float32) + b[...]

    return pl.pallas_call(
        body,
        grid=(n // _BM,),
        in_specs=[_row_spec(d), _row_spec(d), _row_spec(d),
                  _full_spec(wrelT.shape), _full_spec(wrootT.shape), _full_spec(brel.shape)],
        out_specs=[_row_spec(d), _row_spec(d)],
        out_shape=[jax.ShapeDtypeStruct((n, d), jnp.float32)] * 2,
    )(sp[0], sp[1], r0, wrelT, wrootT, brel)


def _stage_c(s1p, r1, sdp, dur_x, gdWrelT, gdWrootT, gdb, gcWrelT, wcombT, bcomb):
    n, d = r1.shape
    dd = dur_x.shape[1]

    def body(sa, sb, r1_ref, da, db, dx, gwr, gwt, gb, cwr, cwt, cb, y_ref, r_ref):
        f2 = jax.nn.relu(sa[...] + sb[...] + r1_ref[...])
        dagg = da[...] + db[...]
        dvec = jax.nn.relu(
            jnp.dot(dagg, gwr[...], preferred_element_type=jnp.float32)
            + jnp.dot(dx[...], gwt[...], preferred_element_type=jnp.float32)
            + gb[...])
        h = jnp.concatenate([f2, dvec], axis=1)
        y_ref[...] = jnp.dot(h, cwr[...], preferred_element_type=jnp.float32)
        r_ref[...] = jnp.dot(h, cwt[...], preferred_element_type=jnp.float32) + cb[...]

    return pl.pallas_call(
        body,
        grid=(n // _BM,),
        in_specs=[_row_spec(d), _row_spec(d), _row_spec(d),
                  _row_spec(dd), _row_spec(dd), _row_spec(dd),
                  _full_spec(gdWrelT.shape), _full_spec(gdWrootT.shape), _full_spec(gdb.shape),
                  _full_spec(gcWrelT.shape), _full_spec(wcombT.shape), _full_spec(bcomb.shape)],
        out_specs=[_row_spec(d), _row_spec(d)],
        out_shape=[jax.ShapeDtypeStruct((n, d), jnp.float32)] * 2,
    )(s1p[0], s1p[1], r1, sdp[0], sdp[1], dur_x,
      gdWrelT, gdWrootT, gdb, gcWrelT, wcombT, bcomb)


def _stage_d(s2p, r2):
    n, d = r2.shape

    def body(sa, sb, r2_ref, ps_ref):
        g = jax.nn.relu(sa[...] + sb[...] + r2_ref[...])
        blk = jnp.sum(g, axis=0, keepdims=True)

        @pl.when(pl.program_id(0) == 0)
        def _():
            ps_ref[...] = blk

        @pl.when(pl.program_id(0) != 0)
        def _():
            ps_ref[...] = ps_ref[...] + blk

    return pl.pallas_call(
        body,
        grid=(n // _BM,),
        in_specs=[_row_spec(d), _row_spec(d), _row_spec(d)],
        out_specs=pl.BlockSpec((1, d), lambda i: (0, 0)),
        out_shape=jax.ShapeDtypeStruct((1, d), jnp.float32),
    )(s2p[0], s2p[1], r2)


def _tail(psum, n_nodes, seqf, fc0WT, fc0b, fccWT, fccb, clsWT, clsb):
    def body(ps, sf, fw, fb, cw, cb, kw, kb, out_ref):
        pooled = ps[...] * (1.0 / n_nodes)
        s = jax.nn.relu(jnp.dot(sf[...], fw[...], preferred_element_type=jnp.float32) + fb[...])
        c = jnp.concatenate([pooled, s], axis=1)
        c = jax.nn.relu(jnp.dot(c, cw[...], preferred_element_type=jnp.float32) + cb[...])
        out_ref[...] = jnp.dot(c, kw[...], preferred_element_type=jnp.float32) + kb[...]

    args = (psum, seqf, fc0WT, fc0b, fccWT, fccb, clsWT, clsb)
    return pl.pallas_call(
        body,
        in_specs=[pl.BlockSpec(a.shape, lambda _nd=a.ndim: (0,) * _nd) for a in args],
        out_specs=pl.BlockSpec((1, clsWT.shape[1]), lambda: (0, 0)),
        out_shape=jax.ShapeDtypeStruct((1, clsWT.shape[1]), jnp.float32),
    )(*args)


def kernel(x, edge_index, edge_attr, dur_x, dur_edge_index, dur_edge_attr,
           sequence_features, g0_Wrel, g0_brel, g0_Wroot, g1_Wrel, g1_brel,
           g1_Wroot, gd0_Wrel, gd0_brel, gd0_Wroot, gc0_Wrel, gc0_brel,
           gc0_Wroot, skip0_W, skip0_b, fc0_W, fc0_b, fcc0_W, fcc0_b,
           cls_W, cls_b):
    n = x.shape[0]
    src = edge_index[0].astype(jnp.int32)
    dst = edge_index[1].astype(jnp.int32)
    dsrc = dur_edge_index[0].astype(jnp.int32)
    ddst = dur_edge_index[1].astype(jnp.int32)
    ea = edge_attr.astype(jnp.float32)
    dea = dur_edge_attr.astype(jnp.float32)

    wcomb = gc0_Wroot + skip0_W
    bcomb = (gc0_brel + skip0_b).reshape(1, -1)

    sdp = _seg_agg(dur_x, dsrc, ddst, dea)

    y0, r0 = _stage_a(x, g0_Wrel.T, g0_Wroot.T, g0_brel.reshape(1, -1))
    s0p = _seg_agg(y0, src, dst, ea)
    y1, r1 = _stage_b(s0p, r0, g1_Wrel.T, g1_Wroot.T, g1_brel.reshape(1, -1))
    s1p = _seg_agg(y1, src, dst, ea)
    y2, r2 = _stage_c(s1p, r1, sdp, dur_x, gd0_Wrel.T, gd0_Wroot.T,
                      gd0_brel.reshape(1, -1), gc0_Wrel.T, wcomb.T, bcomb)
    s2p = _seg_agg(y2, src, dst, ea)
    psum = _stage_d(s2p, r2)
    return _tail(psum, n, sequence_features, fc0_W.T, fc0_b.reshape(1, -1),
                 fcc0_W.T, fcc0_b.reshape(1, -1), cls_W.T, cls_b.reshape(1, -1))
